# Initial kernel scaffold; baseline (speedup 1.0000x reference)
import jax
import jax.numpy as jnp
from jax.experimental import pallas as pl


def kernel(out1, out2, neighbour, W_comb, b_comb, W_agg, b_agg):
    raise NotImplementedError("write your pallas kernel here")


# speedup vs baseline: 1.1977x; 1.1977x over previous
import jax
import jax.numpy as jnp
from jax import lax
from jax.experimental import pallas as pl
from jax.experimental.pallas import tpu as pltpu

_N = 10


def _body(out1_ref, out2_ref, nb_ref, Wc_ref, bc_ref, Wa_ref, ba_ref,
          out3_ref, out4_ref):
    out1 = out1_ref[...]
    out2 = out2_ref[...]
    nb = nb_ref[...]

    a1 = jnp.concatenate([out1, out2], axis=1)
    out3 = lax.dot_general(a1, Wc_ref[...],
                           (((1,), (1,)), ((), ())),
                           preferred_element_type=jnp.float32)
    out3_ref[...] = out3 + bc_ref[...][None, :]

    cols = lax.broadcasted_iota(jnp.int32, (_N, _N), 1)
    counts = jnp.zeros((_N, _N), jnp.float32)
    for k in range(3):
        counts = counts + (nb[:, k][:, None] == cols).astype(jnp.float32)
    eye = (lax.broadcasted_iota(jnp.int32, (_N, _N), 0) == cols)
    A = (counts + eye.astype(jnp.float32)) * 0.25
    vec4 = lax.dot_general(A, out2, (((1,), (0,)), ((), ())),
                           preferred_element_type=jnp.float32)
    out4 = lax.dot_general(vec4, Wa_ref[...],
                           (((1,), (1,)), ((), ())),
                           preferred_element_type=jnp.float32)
    out4_ref[...] = out4 + ba_ref[...][None, :]


def kernel(out1, out2, neighbour, W_comb, b_comb, W_agg, b_agg):
    out3, out4 = pl.pallas_call(
        _body,
        out_shape=(
            jax.ShapeDtypeStruct((_N, 512), jnp.float32),
            jax.ShapeDtypeStruct((_N, 512), jnp.float32),
        ),
    )(out1, out2, neighbour, W_comb, b_comb, W_agg, b_agg)
    return (out3, out4)
